# baseline (speedup 1.0000x reference)
import jax
import jax.numpy as jnp
from jax.experimental import pallas as pl

NB_CODE = 1024
CODE_DIM = 256
QUANT_RESI = 0.5
TLS = [32, 64, 128, 256]
G = 4

_HI = jax.lax.Precision.HIGHEST
_BF = jnp.bfloat16
_F32 = jnp.float32


def _resize_mat(t_in, t_out):
    eye = jnp.eye(t_in, dtype=_F32)
    return jax.image.resize(eye[None], (1, t_in, t_out), method='linear')[0]


def _vq_kernel(x_ref, cb_ref, cbsq_ref,
               wm_ref, b_ref,
               wd0_ref, wd1_ref, wd2_ref, wu0_ref, wu1_ref, wu2_ref,
               out_ref):
    C = CODE_DIM
    K = NB_CODE
    T = x_ref.shape[2]
    GC = G * C

    cb = cb_ref[...]
    cb_bf = cb.astype(_BF)
    cb_hi = cb_bf
    r1s = cb - cb_hi.astype(_F32)
    cb_md = r1s.astype(_BF)
    cb_lo = (r1s - cb_md.astype(_F32)).astype(_BF)
    cbsq = cbsq_ref[...]

    r = x_ref[...].reshape(GC, T)
    f = jnp.zeros((GC, T), _F32)
    wds = [wd0_ref, wd1_ref, wd2_ref, None]
    wus = [wu0_ref, wu1_ref, wu2_ref, None]

    for si in range(4):
        tl = TLS[si]
        if tl != T:
            zst = jax.lax.dot_general(r, wds[si][...],
                                      (((1,), (0,)), ((), ())),
                                      precision=_HI,
                                      preferred_element_type=_F32)
        else:
            zst = r
        gtl = G * tl
        score = jnp.concatenate(
            [jax.lax.dot_general(zst[g * C:(g + 1) * C].astype(_BF), cb_bf,
                                 (((0,), (1,)), ((), ())),
                                 preferred_element_type=_F32)
             for g in range(G)], axis=0)
        negdist = 2.0 * score - cbsq
        m = jnp.max(negdist, axis=1, keepdims=True)
        iota = jax.lax.broadcasted_iota(jnp.int32, (gtl, K), 1)
        idx = jnp.min(jnp.where(negdist == m, iota, K), axis=1)
        onehot = (iota == idx[:, None]).astype(_BF)
        parts = [jax.lax.dot_general(p, onehot,
                                     (((0,), (1,)), ((), ())),
                                     preferred_element_type=_F32)
                 for p in (cb_hi, cb_md, cb_lo)]
        zq_cat = (parts[0] + parts[1]) + parts[2]
        zq_st = jnp.concatenate(
            [zq_cat[:, g * tl:(g + 1) * tl] for g in range(G)], axis=0)
        if tl != T:
            h_pre = jax.lax.dot_general(zq_st, wus[si][...],
                                        (((1,), (0,)), ((), ())),
                                        precision=_HI,
                                        preferred_element_type=_F32)
        else:
            h_pre = zq_st
        zcol = jnp.zeros((GC, 1), _F32)
        hs0 = jnp.concatenate([zcol, h_pre[:, :-1]], axis=1)
        hs2 = jnp.concatenate([h_pre[:, 1:], zcol], axis=1)
        wmb = [wm_ref[si, k].astype(_BF) for k in range(3)]
        hsb = [s.astype(_BF) for s in (hs0, h_pre, hs2)]
        bias = b_ref[si][:, None]
        conv = jnp.concatenate(
            [(jax.lax.dot_general(wmb[0], hsb[0][g * C:(g + 1) * C],
                                  (((1,), (0,)), ((), ())),
                                  preferred_element_type=_F32)
              + jax.lax.dot_general(wmb[1], hsb[1][g * C:(g + 1) * C],
                                    (((1,), (0,)), ((), ())),
                                    preferred_element_type=_F32)
              + jax.lax.dot_general(wmb[2], hsb[2][g * C:(g + 1) * C],
                                    (((1,), (0,)), ((), ())),
                                    preferred_element_type=_F32))
             + bias
             for g in range(G)], axis=0)
        h = h_pre * (1.0 - QUANT_RESI) + conv * QUANT_RESI
        f = f + h
        r = r - h

    out_ref[...] = f.reshape(G, C, T)


@jax.jit
def kernel(x, codebook, phi_w, phi_b):
    N, C, T = x.shape
    cbsq = jnp.sum(codebook ** 2, axis=1)[None, :]
    wd = [_resize_mat(T, tl) for tl in TLS[:3]]
    wu = [_resize_mat(tl, T) for tl in TLS[:3]]
    wm = jnp.transpose(phi_w, (0, 3, 1, 2))

    rep = lambda *shape: pl.BlockSpec(shape, lambda n: (0,) * len(shape))
    grid_spec = pl.GridSpec(
        grid=(N // G,),
        in_specs=[
            pl.BlockSpec((G, C, T), lambda n: (n, 0, 0)),
            rep(NB_CODE, C),
            rep(1, NB_CODE),
            rep(4, 3, C, C),
            rep(4, C),
            rep(T, TLS[0]), rep(T, TLS[1]), rep(T, TLS[2]),
            rep(TLS[0], T), rep(TLS[1], T), rep(TLS[2], T),
        ],
        out_specs=pl.BlockSpec((G, C, T), lambda n: (n, 0, 0)),
    )
    return pl.pallas_call(
        _vq_kernel,
        grid_spec=grid_spec,
        out_shape=jax.ShapeDtypeStruct((N, C, T), _F32),
    )(x, codebook, cbsq, wm, phi_b, *wd, *wu)


# speedup vs baseline: 1.5558x; 1.5558x over previous
import jax
import jax.numpy as jnp
from jax.experimental import pallas as pl

NB_CODE = 1024
CODE_DIM = 256
QUANT_RESI = 0.5
TLS = [32, 64, 128, 256]
G = 8

_HI = jax.lax.Precision.HIGHEST
_BF = jnp.bfloat16
_F32 = jnp.float32


def _resize_mat(t_in, t_out):
    eye = jnp.eye(t_in, dtype=_F32)
    return jax.image.resize(eye[None], (1, t_in, t_out), method='linear')[0]


def _vq_kernel(x_ref, cb_ref, cbsq_ref,
               wm_ref, b_ref,
               wd0_ref, wd1_ref, wd2_ref, wu0_ref, wu1_ref, wu2_ref,
               m0_ref, m2_ref,
               out_ref):
    C = CODE_DIM
    K = NB_CODE
    T = x_ref.shape[2]
    GT = G * T

    cb = cb_ref[...]
    cb_bf = cb.astype(_BF)
    cb_hi = cb_bf
    r1s = cb - cb_hi.astype(_F32)
    cb_md = r1s.astype(_BF)
    cb_lo = (r1s - cb_md.astype(_F32)).astype(_BF)
    cbsq = cbsq_ref[...]
    mask0 = m0_ref[...]
    mask2 = m2_ref[...]

    r = jnp.concatenate([x_ref[g] for g in range(G)], axis=1)
    f = jnp.zeros((C, GT), _F32)
    wds = [wd0_ref, wd1_ref, wd2_ref, None]
    wus = [wu0_ref, wu1_ref, wu2_ref, None]

    for si in range(4):
        tl = TLS[si]
        if tl != T:
            z_cat = jnp.concatenate(
                [jax.lax.dot_general(r[:, g * T:(g + 1) * T], wds[si][...],
                                     (((1,), (0,)), ((), ())),
                                     precision=_HI,
                                     preferred_element_type=_F32)
                 for g in range(G)], axis=1)
        else:
            z_cat = r
        gtl = G * tl
        score = jax.lax.dot_general(z_cat.astype(_BF), cb_bf,
                                    (((0,), (1,)), ((), ())),
                                    preferred_element_type=_F32)
        negdist = 2.0 * score - cbsq
        m = jnp.max(negdist, axis=1, keepdims=True)
        iota = jax.lax.broadcasted_iota(jnp.int32, (gtl, K), 1)
        idx = jnp.min(jnp.where(negdist == m, iota, K), axis=1)
        onehot = (iota == idx[:, None]).astype(_BF)
        parts = [jax.lax.dot_general(p, onehot,
                                     (((0,), (1,)), ((), ())),
                                     preferred_element_type=_F32)
                 for p in (cb_hi, cb_md, cb_lo)]
        zq_cat = (parts[0] + parts[1]) + parts[2]
        if tl != T:
            h_pre = jnp.concatenate(
                [jax.lax.dot_general(zq_cat[:, g * tl:(g + 1) * tl],
                                     wus[si][...],
                                     (((1,), (0,)), ((), ())),
                                     precision=_HI,
                                     preferred_element_type=_F32)
                 for g in range(G)], axis=1)
        else:
            h_pre = zq_cat
        zcol = jnp.zeros((C, 1), _F32)
        hs0 = jnp.concatenate([zcol, h_pre[:, :-1]], axis=1) * mask0
        hs2 = jnp.concatenate([h_pre[:, 1:], zcol], axis=1) * mask2
        es = [jax.lax.dot_general(wm_ref[si, k].astype(_BF), s.astype(_BF),
                                  (((1,), (0,)), ((), ())),
                                  preferred_element_type=_F32)
              for k, s in enumerate((hs0, h_pre, hs2))]
        conv = (es[0] + es[1] + es[2]) + b_ref[si][:, None]
        h = h_pre * (1.0 - QUANT_RESI) + conv * QUANT_RESI
        f = f + h
        r = r - h

    for g in range(G):
        out_ref[g] = f[:, g * T:(g + 1) * T]


@jax.jit
def kernel(x, codebook, phi_w, phi_b):
    N, C, T = x.shape
    GT = G * T
    cbsq = jnp.sum(codebook ** 2, axis=1)[None, :]
    wd = [_resize_mat(T, tl) for tl in TLS[:3]]
    wu = [_resize_mat(tl, T) for tl in TLS[:3]]
    wm = jnp.transpose(phi_w, (0, 3, 1, 2))
    tpos = jnp.arange(GT) % T
    mask0 = (tpos != 0).astype(_F32)[None, :]
    mask2 = (tpos != T - 1).astype(_F32)[None, :]

    rep = lambda *shape: pl.BlockSpec(shape, lambda n: (0,) * len(shape))
    grid_spec = pl.GridSpec(
        grid=(N // G,),
        in_specs=[
            pl.BlockSpec((G, C, T), lambda n: (n, 0, 0)),
            rep(NB_CODE, C),
            rep(1, NB_CODE),
            rep(4, 3, C, C),
            rep(4, C),
            rep(T, TLS[0]), rep(T, TLS[1]), rep(T, TLS[2]),
            rep(TLS[0], T), rep(TLS[1], T), rep(TLS[2], T),
            rep(1, GT), rep(1, GT),
        ],
        out_specs=pl.BlockSpec((G, C, T), lambda n: (n, 0, 0)),
    )
    return pl.pallas_call(
        _vq_kernel,
        grid_spec=grid_spec,
        out_shape=jax.ShapeDtypeStruct((N, C, T), _F32),
    )(x, codebook, cbsq, wm, phi_b, *wd, *wu, mask0, mask2)
